# baseline (device time: 17906 ns/iter reference)
import jax
import jax.numpy as jnp
from jax import lax
from jax.experimental import pallas as pl
from jax.experimental.pallas import tpu as pltpu

M = 2048
N = 1024
N_HALF = N // 2
N_QTR = N_HALF // 2
C = 16
CK = M // C
SCALE = 5.0 / 127.0


def kernel(x):
    def body(
        x_hbm,
        out_hbm,
        xv,
        ov,
        q_send,
        q_recv,
        in_sems,
        out_sems,
        send_sems,
        recv_sems,
    ):
        my_x = lax.axis_index("x")
        my_y = lax.axis_index("y")
        my_z = lax.axis_index("z")
        peer_y = 1 - my_y

        in_dmas = []
        for i in range(C):
            rows = pl.ds(i * CK, CK)
            d = pltpu.make_async_copy(x_hbm.at[0, rows, :], xv.at[rows], in_sems.at[i])
            d.start()
            in_dmas.append(d)

        barrier_sem = pltpu.get_barrier_semaphore()
        pl.semaphore_signal(
            barrier_sem,
            inc=1,
            device_id=(my_x, peer_y, my_z),
            device_id_type=pl.DeviceIdType.MESH,
        )
        pl.semaphore_wait(barrier_sem, 1)

        def make_rdma(i):
            return pltpu.make_async_remote_copy(
                src_ref=q_send.at[pl.ds(i * CK, CK)],
                dst_ref=q_recv.at[pl.ds(i * CK, CK)],
                send_sem=send_sems.at[i],
                recv_sem=recv_sems.at[i],
                device_id=(my_x, peer_y, my_z),
                device_id_type=pl.DeviceIdType.MESH,
            )

        def quant(v):
            return jnp.clip(
                jnp.round(v * (1.0 / SCALE)), -127.0, 127.0
            ).astype(jnp.int32)

        def quant_pack_chunk(i, send_c0):
            rows = pl.ds(i * CK, CK)
            qa = quant(xv[rows, send_c0 : send_c0 + N_QTR])
            qb = quant(xv[rows, send_c0 + N_QTR : send_c0 + N_HALF])
            packed = (qb << 8) | (qa & 0xFF)
            q_send[rows] = packed.astype(jnp.int16)

        def unpack_add_chunk(i, my_c0):
            rows = pl.ds(i * CK, CK)
            p = q_recv[rows].astype(jnp.int32)
            qa = (p << 24) >> 24
            qb = p >> 8
            ov[rows, 0:N_QTR] = (
                xv[rows, my_c0 : my_c0 + N_QTR] + qa.astype(jnp.float32) * SCALE
            ).astype(jnp.bfloat16)
            ov[rows, N_QTR:N_HALF] = (
                xv[rows, my_c0 + N_QTR : my_c0 + N_HALF]
                + qb.astype(jnp.float32) * SCALE
            ).astype(jnp.bfloat16)

        rdmas = [make_rdma(i) for i in range(C)]
        out_dmas = [
            pltpu.make_async_copy(
                ov.at[pl.ds(i * CK, CK)],
                out_hbm.at[pl.ds(i * CK, CK)],
                out_sems.at[i],
            )
            for i in range(C)
        ]

        for my_c0, send_c0, y_val in ((0, N_HALF, 0), (N_HALF, 0, 1)):

            @pl.when(my_y == y_val)
            def _(my_c0=my_c0, send_c0=send_c0):
                for i in range(C):
                    in_dmas[i].wait()
                    quant_pack_chunk(i, send_c0)
                    rdmas[i].start()
                for i in range(C):
                    rdmas[i].wait_recv()
                    unpack_add_chunk(i, my_c0)
                    out_dmas[i].start()

        for i in range(C):
            out_dmas[i].wait()
            rdmas[i].wait_send()

    return pl.pallas_call(
        body,
        out_shape=jax.ShapeDtypeStruct((M, N_HALF), jnp.bfloat16),
        in_specs=[pl.BlockSpec(memory_space=pltpu.MemorySpace.HBM)],
        out_specs=pl.BlockSpec(memory_space=pltpu.MemorySpace.HBM),
        scratch_shapes=[
            pltpu.VMEM((M, N), jnp.float32),
            pltpu.VMEM((M, N_HALF), jnp.bfloat16),
            pltpu.VMEM((M, N_QTR), jnp.int16),
            pltpu.VMEM((M, N_QTR), jnp.int16),
            pltpu.SemaphoreType.DMA((C,)),
            pltpu.SemaphoreType.DMA((C,)),
            pltpu.SemaphoreType.DMA((C,)),
            pltpu.SemaphoreType.DMA((C,)),
        ],
        compiler_params=pltpu.CompilerParams(collective_id=0),
    )(pltpu.with_memory_space_constraint(x, pltpu.MemorySpace.HBM))


# device time: 17441 ns/iter; 1.0267x vs baseline; 1.0267x over previous
import jax
import jax.numpy as jnp
from jax import lax
from jax.experimental import pallas as pl
from jax.experimental.pallas import tpu as pltpu

M = 2048
N = 1024
N_HALF = N // 2
N_QTR = N_HALF // 2
CHUNK_ROWS = (64, 128, 256, 256, 256, 256, 256, 256, 256, 64)
assert sum(CHUNK_ROWS) == M
CHUNK_OFF = tuple(sum(CHUNK_ROWS[:i]) for i in range(len(CHUNK_ROWS)))
C = len(CHUNK_ROWS)
SCALE = 5.0 / 127.0


def kernel(x):
    def body(
        x_hbm,
        out_hbm,
        xv,
        ov,
        q_send,
        q_recv,
        in_sems,
        out_sems,
        send_sems,
        recv_sems,
    ):
        my_x = lax.axis_index("x")
        my_y = lax.axis_index("y")
        my_z = lax.axis_index("z")
        peer_y = 1 - my_y

        in_dmas = []
        for i in range(C):
            rows = pl.ds(CHUNK_OFF[i], CHUNK_ROWS[i])
            d = pltpu.make_async_copy(x_hbm.at[0, rows, :], xv.at[rows], in_sems.at[i])
            d.start()
            in_dmas.append(d)

        barrier_sem = pltpu.get_barrier_semaphore()
        pl.semaphore_signal(
            barrier_sem,
            inc=1,
            device_id=(my_x, peer_y, my_z),
            device_id_type=pl.DeviceIdType.MESH,
        )
        pl.semaphore_wait(barrier_sem, 1)

        def make_rdma(i):
            return pltpu.make_async_remote_copy(
                src_ref=q_send.at[pl.ds(CHUNK_OFF[i], CHUNK_ROWS[i])],
                dst_ref=q_recv.at[pl.ds(CHUNK_OFF[i], CHUNK_ROWS[i])],
                send_sem=send_sems.at[i],
                recv_sem=recv_sems.at[i],
                device_id=(my_x, peer_y, my_z),
                device_id_type=pl.DeviceIdType.MESH,
            )

        def quant(v):
            return jnp.clip(
                jnp.round(v * (1.0 / SCALE)), -127.0, 127.0
            ).astype(jnp.int32)

        def quant_pack_chunk(i, send_c0):
            rows = pl.ds(CHUNK_OFF[i], CHUNK_ROWS[i])
            qa = quant(xv[rows, send_c0 : send_c0 + N_QTR])
            qb = quant(xv[rows, send_c0 + N_QTR : send_c0 + N_HALF])
            packed = (qb << 8) | (qa & 0xFF)
            q_send[rows] = packed.astype(jnp.int16)

        def unpack_add_chunk(i, my_c0):
            rows = pl.ds(CHUNK_OFF[i], CHUNK_ROWS[i])
            p = q_recv[rows].astype(jnp.int32)
            qa = (p << 24) >> 24
            qb = p >> 8
            ov[rows, 0:N_QTR] = (
                xv[rows, my_c0 : my_c0 + N_QTR] + qa.astype(jnp.float32) * SCALE
            ).astype(jnp.bfloat16)
            ov[rows, N_QTR:N_HALF] = (
                xv[rows, my_c0 + N_QTR : my_c0 + N_HALF]
                + qb.astype(jnp.float32) * SCALE
            ).astype(jnp.bfloat16)

        rdmas = [make_rdma(i) for i in range(C)]
        out_dmas = [
            pltpu.make_async_copy(
                ov.at[pl.ds(CHUNK_OFF[i], CHUNK_ROWS[i])],
                out_hbm.at[pl.ds(CHUNK_OFF[i], CHUNK_ROWS[i])],
                out_sems.at[i],
            )
            for i in range(C)
        ]

        for my_c0, send_c0, y_val in ((0, N_HALF, 0), (N_HALF, 0, 1)):

            @pl.when(my_y == y_val)
            def _(my_c0=my_c0, send_c0=send_c0):
                for i in range(C):
                    in_dmas[i].wait()
                    quant_pack_chunk(i, send_c0)
                    rdmas[i].start()
                for i in range(C):
                    rdmas[i].wait_recv()
                    unpack_add_chunk(i, my_c0)
                    out_dmas[i].start()

        for i in range(C):
            out_dmas[i].wait()
            rdmas[i].wait_send()

    return pl.pallas_call(
        body,
        out_shape=jax.ShapeDtypeStruct((M, N_HALF), jnp.bfloat16),
        in_specs=[pl.BlockSpec(memory_space=pltpu.MemorySpace.HBM)],
        out_specs=pl.BlockSpec(memory_space=pltpu.MemorySpace.HBM),
        scratch_shapes=[
            pltpu.VMEM((M, N), jnp.float32),
            pltpu.VMEM((M, N_HALF), jnp.bfloat16),
            pltpu.VMEM((M, N_QTR), jnp.int16),
            pltpu.VMEM((M, N_QTR), jnp.int16),
            pltpu.SemaphoreType.DMA((C,)),
            pltpu.SemaphoreType.DMA((C,)),
            pltpu.SemaphoreType.DMA((C,)),
            pltpu.SemaphoreType.DMA((C,)),
        ],
        compiler_params=pltpu.CompilerParams(collective_id=0),
    )(pltpu.with_memory_space_constraint(x, pltpu.MemorySpace.HBM))
